# Initial kernel scaffold; baseline (speedup 1.0000x reference)
import jax
import jax.numpy as jnp
from jax.experimental import pallas as pl


def kernel(x_wave, x_transition, x_target, t_wave, t_transition, t_target, edge_index_wt, edge_index_tt, params):
    raise NotImplementedError("write your pallas kernel here")


# speedup vs baseline: 1.0908x; 1.0908x over previous
import functools

import jax
import jax.numpy as jnp
from jax.experimental import pallas as pl
from jax.experimental.pallas import tpu as pltpu

D = 128
H = 4
HD = D // H
_INV_SQRT_HD = 1.0 / (HD ** 0.5)


def _dense_body(xs_ref, xt_ref, ts_ref, wsrc_ref, bsrc_ref, wdst_ref, bdst_ref, wt_ref,
                src_ref, dst_ref, tm_ref):
    xs = xs_ref[...]
    xt = xt_ref[...]
    ts = ts_ref[...]
    src_ref[...] = jnp.dot(xs, wsrc_ref[...], preferred_element_type=jnp.float32) + bsrc_ref[...]
    dst_ref[...] = jnp.dot(xt, wdst_ref[...], preferred_element_type=jnp.float32) + bdst_ref[...]
    tm_ref[...] = jnp.dot(ts, wt_ref[...], preferred_element_type=jnp.float32)


def _dense_stage(xs, xt, ts, p):
    n = xs.shape[0]
    w1a = p['mlp1_w'][:D, :]
    w1b = p['mlp1_w'][D:, :]
    wsrc = jnp.concatenate([w1a, p['eq_w'], p['Wk'], p['Wv']], axis=1)
    bsrc = jnp.concatenate([jnp.zeros((D,), jnp.float32), p['eq_b'],
                            jnp.zeros((2 * D,), jnp.float32)])[None, :]
    wdst = jnp.concatenate([w1b, p['ek_w'], p['Wq']], axis=1)
    bdst = jnp.concatenate([p['mlp1_b'], p['ek_b'], jnp.zeros((D,), jnp.float32)])[None, :]

    blk = 400
    grid = (n // blk,)
    src, dst, tm = pl.pallas_call(
        _dense_body,
        grid=grid,
        in_specs=[
            pl.BlockSpec((blk, D), lambda i: (i, 0)),
            pl.BlockSpec((blk, D), lambda i: (i, 0)),
            pl.BlockSpec((blk, D), lambda i: (i, 0)),
            pl.BlockSpec((D, 4 * D), lambda i: (0, 0)),
            pl.BlockSpec((1, 4 * D), lambda i: (0, 0)),
            pl.BlockSpec((D, 3 * D), lambda i: (0, 0)),
            pl.BlockSpec((1, 3 * D), lambda i: (0, 0)),
            pl.BlockSpec((D, D), lambda i: (0, 0)),
        ],
        out_specs=[
            pl.BlockSpec((blk, 4 * D), lambda i: (i, 0)),
            pl.BlockSpec((blk, 3 * D), lambda i: (i, 0)),
            pl.BlockSpec((blk, D), lambda i: (i, 0)),
        ],
        out_shape=[
            jax.ShapeDtypeStruct((n, 4 * D), jnp.float32),
            jax.ShapeDtypeStruct((n, 3 * D), jnp.float32),
            jax.ShapeDtypeStruct((n, D), jnp.float32),
        ],
    )(xs, xt, ts, wsrc, bsrc, wdst, bdst, p['Wt'])
    return src, dst, tm


def _conv(xs, xt, ts, tt, ei, p):
    src = ei[0]
    dst = ei[1]
    n_t = xt.shape[0]
    srctab, dsttab, tm = _dense_stage(xs, xt, ts, p)

    gs = srctab[src]
    gd = dsttab[dst]
    a = gs[:, :D]
    qe = gs[:, D:2 * D].reshape(-1, H, HD)
    k = gs[:, 2 * D:3 * D].reshape(-1, H, HD)
    v = gs[:, 3 * D:]
    b = gd[:, :D]
    ke = gd[:, D:2 * D].reshape(-1, H, HD)
    q = gd[:, 2 * D:].reshape(-1, H, HD)

    hmid = jax.nn.relu(a + b)
    mlp_w = jax.nn.sigmoid(hmid @ p['mlp2_w'] + p['mlp2_b'])
    s_e = jnp.sum(qe * ke, axis=-1) * _INV_SQRT_HD
    aw = jax.nn.sigmoid(s_e)
    z = p['sw'] * mlp_w + p['fw'] * aw
    pz = jnp.exp(z)
    zsum = jnp.sum(pz, axis=0)
    ew = pz / zsum
    scores = jnp.sum(q * k, axis=-1) * _INV_SQRT_HD
    ws = scores * ew
    e = jnp.exp(ws)
    s_seg = jax.ops.segment_sum(e, dst, num_segments=n_t)
    deg = jax.ops.segment_sum(jnp.ones((e.shape[0],), jnp.float32), dst, num_segments=n_t)
    u = jax.ops.segment_sum((e[..., None] * v.reshape(-1, H, HD)).reshape(-1, D), dst,
                            num_segments=n_t)
    attn_den = jnp.repeat(s_seg + 1e-16, HD, axis=1)
    x_out = (u / attn_den) @ p['Wout_w'] + deg[:, None] * p['Wout_b'][None, :]
    t_out = jax.ops.segment_sum(tm[src], dst, num_segments=n_t)
    return x_out, t_out


def _layer_norm(x, g, b):
    mu = jnp.mean(x, axis=-1, keepdims=True)
    var = jnp.var(x, axis=-1, keepdims=True)
    return (x - mu) / jnp.sqrt(var + 1e-5) * g + b


def kernel(x_wave, x_transition, x_target, t_wave, t_transition, t_target,
           edge_index_wt, edge_index_tt, params):
    p = params
    xu, tu = _conv(x_wave, x_transition, t_wave, t_transition, edge_index_wt, p['c1'])
    x_trans = _layer_norm(p['rw'] * x_transition + p['ewa'] * jax.nn.relu(xu), p['ln_g'], p['ln_b'])
    t_trans = _layer_norm(p['rw'] * t_transition + p['ewa'] * jax.nn.relu(tu), p['ln_g'], p['ln_b'])
    xu2, tu2 = _conv(x_trans, x_target, t_trans, t_target, edge_index_tt, p['c2'])
    x_tgt = _layer_norm(p['rw'] * x_target + p['ewa'] * jax.nn.relu(xu2), p['ln_g'], p['ln_b'])
    t_tgt = _layer_norm(p['rw'] * t_target + p['ewa'] * jax.nn.relu(tu2), p['ln_g'], p['ln_b'])
    return (x_wave, x_trans, x_tgt, t_wave, t_trans, t_tgt)
